# Initial kernel scaffold; baseline (speedup 1.0000x reference)
import jax
import jax.numpy as jnp
from jax.experimental import pallas as pl


def kernel(x, shared, mlp_w1, mlp_b1, mlp_w2, mlp_b2, gate_w, freq_gate_w, p0, p1, p2, proj_out_w):
    raise NotImplementedError("write your pallas kernel here")


# speedup vs baseline: 3.3631x; 3.3631x over previous
import functools

import jax
import jax.numpy as jnp
from jax.experimental import pallas as pl
from jax.experimental.pallas import tpu as pltpu

DIM = 384
RANK = 96
E = 8
K = 2
H = 224
W = 224
P = H * W

C_BLK = 64
T_PIX = 1792

_INV_SQRT2 = 0.7071067811865476


def _gelu_exact(x):
    return 0.5 * x * (1.0 + jax.lax.erf(x * _INV_SQRT2))


def _reduce_body(x_ref, fe_ref, pooled_ref):
    xb = x_ref[...]
    c, h, w = xb.shape
    zrow = jnp.zeros((c, 1, w), dtype=xb.dtype)
    up = jnp.concatenate([xb[:, 1:, :], zrow], axis=1)
    dn = jnp.concatenate([zrow, xb[:, :-1, :]], axis=1)
    sv = xb + up + dn
    zcol = jnp.zeros((c, h, 1), dtype=xb.dtype)
    lf = jnp.concatenate([sv[:, :, 1:], zcol], axis=2)
    rt = jnp.concatenate([zcol, sv[:, :, :-1]], axis=2)
    box = sv + lf + rt
    hp = 9.0 * xb - box
    ge = _gelu_exact(hp)
    fe_ref[...] = jnp.sum(ge, axis=1)
    pooled_ref[...] = jnp.sum(xb, axis=1)


def _router_body(fe_ref, pooled_ref, w1_ref, b1_ref, w2_ref, b2_ref,
                 gate_ref, freq_ref, idx_ref, vals_ref):
    inv = 1.0 / float(P)
    fe0 = jnp.sum(fe_ref[...], axis=1, keepdims=True) * inv
    pooled = jnp.sum(pooled_ref[...], axis=1, keepdims=True) * inv
    h1 = jnp.dot(w1_ref[...], fe0, preferred_element_type=jnp.float32)
    h1 = _gelu_exact(h1 + b1_ref[...])
    fe2 = jnp.dot(w2_ref[...], h1, preferred_element_type=jnp.float32)
    fe2 = fe2 + b2_ref[...]
    logits = (jnp.dot(gate_ref[...], pooled, preferred_element_type=jnp.float32)
              + jnp.dot(freq_ref[...], fe2, preferred_element_type=jnp.float32))
    m = jnp.max(logits, axis=0, keepdims=True)
    ex = jnp.exp(logits - m)
    s = ex / jnp.sum(ex, axis=0, keepdims=True)
    row = jax.lax.broadcasted_iota(jnp.int32, (E, 1), 0)
    v0 = jnp.max(s, axis=0, keepdims=True)
    idx0 = jnp.min(jnp.where(s == v0, row, E), axis=0, keepdims=True)
    s_masked = jnp.where(row == idx0, -jnp.inf, s)
    v1 = jnp.max(s_masked, axis=0, keepdims=True)
    idx1 = jnp.min(jnp.where(s_masked == v1, row, E), axis=0, keepdims=True)
    lane = jax.lax.broadcasted_iota(jnp.int32, (1, 128), 1)
    idx_ref[...] = jnp.where(lane == 0, idx0, jnp.where(lane == 1, idx1, 0))
    vals_ref[...] = jnp.where(lane == 0, v0, jnp.where(lane == 1, v1, 0.0))


def _main_body(idx_ref, vals_ref, x_ref, s_ref, p0a_ref, p0b_ref,
               p1a_ref, p1b_ref, p2a_ref, p2b_ref, proj_ref, out_ref,
               a_s, b_s, c_s):
    @pl.when(pl.program_id(0) == 0)
    def _init():
        a_s[...] = jnp.concatenate([p0a_ref[0], p0b_ref[0]], axis=0)
        b_s[...] = jnp.concatenate([p1a_ref[0], p1b_ref[0]], axis=0)
        c_s[...] = jnp.concatenate(
            [p2a_ref[0] * vals_ref[0], p2b_ref[0] * vals_ref[1]], axis=1)

    xt = x_ref[...]
    h = jnp.dot(a_s[...], xt, preferred_element_type=jnp.float32)
    gg = jnp.dot(b_s[...], s_ref[...], preferred_element_type=jnp.float32)
    g = gg * jax.nn.sigmoid(gg)
    u = jnp.dot(c_s[...], h * g, preferred_element_type=jnp.float32)
    gs = vals_ref[0] + vals_ref[1]
    out_ref[...] = jnp.dot(proj_ref[...], u + gs * xt,
                           preferred_element_type=jnp.float32)


@functools.partial(jax.jit, static_argnames=("interpret",))
def kernel(x, shared, mlp_w1, mlp_b1, mlp_w2, mlp_b2, gate_w, freq_gate_w,
           p0, p1, p2, proj_out_w, interpret=False):
    f32 = jnp.float32
    x3 = x.reshape(DIM, H, W)

    fe_part, pooled_part = pl.pallas_call(
        _reduce_body,
        grid=(DIM // C_BLK,),
        in_specs=[pl.BlockSpec((C_BLK, H, W), lambda i: (i, 0, 0))],
        out_specs=[pl.BlockSpec((C_BLK, W), lambda i: (i, 0)),
                   pl.BlockSpec((C_BLK, W), lambda i: (i, 0))],
        out_shape=[jax.ShapeDtypeStruct((DIM, W), f32),
                   jax.ShapeDtypeStruct((DIM, W), f32)],
        interpret=interpret,
    )(x3)

    idxv, valsv = pl.pallas_call(
        _router_body,
        out_shape=[jax.ShapeDtypeStruct((1, 128), jnp.int32),
                   jax.ShapeDtypeStruct((1, 128), f32)],
        interpret=interpret,
    )(fe_part, pooled_part, mlp_w1, mlp_b1.reshape(2 * DIM, 1),
      mlp_w2, mlp_b2.reshape(DIM, 1), gate_w, freq_gate_w)

    idx = idxv[0, :K]
    vals = valsv[0, :K]

    x2 = x.reshape(DIM, P)
    s2 = shared.reshape(DIM, P)

    grid_spec = pltpu.PrefetchScalarGridSpec(
        num_scalar_prefetch=2,
        grid=(P // T_PIX,),
        in_specs=[
            pl.BlockSpec((DIM, T_PIX), lambda p, i, v: (0, p)),
            pl.BlockSpec((DIM, T_PIX), lambda p, i, v: (0, p)),
            pl.BlockSpec((1, RANK, DIM), lambda p, i, v: (i[0], 0, 0)),
            pl.BlockSpec((1, RANK, DIM), lambda p, i, v: (i[1], 0, 0)),
            pl.BlockSpec((1, RANK, DIM), lambda p, i, v: (i[0], 0, 0)),
            pl.BlockSpec((1, RANK, DIM), lambda p, i, v: (i[1], 0, 0)),
            pl.BlockSpec((1, DIM, RANK), lambda p, i, v: (i[0], 0, 0)),
            pl.BlockSpec((1, DIM, RANK), lambda p, i, v: (i[1], 0, 0)),
            pl.BlockSpec((DIM, DIM), lambda p, i, v: (0, 0)),
        ],
        out_specs=pl.BlockSpec((DIM, T_PIX), lambda p, i, v: (0, p)),
        scratch_shapes=[
            pltpu.VMEM((K * RANK, DIM), f32),
            pltpu.VMEM((K * RANK, DIM), f32),
            pltpu.VMEM((DIM, K * RANK), f32),
        ],
    )

    out2 = pl.pallas_call(
        _main_body,
        grid_spec=grid_spec,
        out_shape=jax.ShapeDtypeStruct((DIM, P), f32),
        interpret=interpret,
    )(idx, vals, x2, s2, p0, p0, p1, p1, p2, p2, proj_out_w)

    return out2.reshape(1, DIM, H, W)


# speedup vs baseline: 3.3750x; 1.0035x over previous
import functools

import jax
import jax.numpy as jnp
from jax.experimental import pallas as pl
from jax.experimental.pallas import tpu as pltpu

DIM = 384
RANK = 96
E = 8
K = 2
H = 224
W = 224
P = H * W

C_BLK = 64
H_BLK = 8

_INV_SQRT2 = 0.7071067811865476


def _gelu_exact(x):
    return 0.5 * x * (1.0 + jax.lax.erf(x * _INV_SQRT2))


def _reduce_body(x_ref, fe_ref, pooled_ref):
    xb = x_ref[...]
    c, h, w = xb.shape
    zrow = jnp.zeros((c, 1, w), dtype=xb.dtype)
    up = jnp.concatenate([xb[:, 1:, :], zrow], axis=1)
    dn = jnp.concatenate([zrow, xb[:, :-1, :]], axis=1)
    sv = xb + up + dn
    zcol = jnp.zeros((c, h, 1), dtype=xb.dtype)
    lf = jnp.concatenate([sv[:, :, 1:], zcol], axis=2)
    rt = jnp.concatenate([zcol, sv[:, :, :-1]], axis=2)
    box = sv + lf + rt
    hp = 9.0 * xb - box
    ge = _gelu_exact(hp)
    fe_ref[...] = jnp.sum(ge, axis=1)
    pooled_ref[...] = jnp.sum(xb, axis=1)


def _router_body(fe_ref, pooled_ref, w1_ref, b1_ref, w2_ref, b2_ref,
                 gate_ref, freq_ref, idx_ref, vals_ref):
    inv = 1.0 / float(P)
    fe0 = jnp.sum(fe_ref[...], axis=1, keepdims=True) * inv
    pooled = jnp.sum(pooled_ref[...], axis=1, keepdims=True) * inv
    h1 = jnp.dot(w1_ref[...], fe0, preferred_element_type=jnp.float32)
    h1 = _gelu_exact(h1 + b1_ref[...])
    fe2 = jnp.dot(w2_ref[...], h1, preferred_element_type=jnp.float32)
    fe2 = fe2 + b2_ref[...]
    logits = (jnp.dot(gate_ref[...], pooled, preferred_element_type=jnp.float32)
              + jnp.dot(freq_ref[...], fe2, preferred_element_type=jnp.float32))
    m = jnp.max(logits, axis=0, keepdims=True)
    ex = jnp.exp(logits - m)
    s = ex / jnp.sum(ex, axis=0, keepdims=True)
    row = jax.lax.broadcasted_iota(jnp.int32, (E, 1), 0)
    v0 = jnp.max(s, axis=0, keepdims=True)
    idx0 = jnp.min(jnp.where(s == v0, row, E), axis=0, keepdims=True)
    s_masked = jnp.where(row == idx0, -jnp.inf, s)
    v1 = jnp.max(s_masked, axis=0, keepdims=True)
    idx1 = jnp.min(jnp.where(s_masked == v1, row, E), axis=0, keepdims=True)
    lane = jax.lax.broadcasted_iota(jnp.int32, (1, 128), 1)
    idx_ref[...] = jnp.where(lane == 0, idx0, jnp.where(lane == 1, idx1, 0))
    vals_ref[...] = jnp.where(lane == 0, v0, jnp.where(lane == 1, v1, 0.0))


def _main_body(idx_ref, vals_ref, x_ref, s_ref, p0a_ref, p0b_ref,
               p1a_ref, p1b_ref, p2a_ref, p2b_ref, proj_ref, out_ref,
               a_s, b_s, c_s):
    @pl.when(pl.program_id(0) == 0)
    def _init():
        a_s[...] = jnp.concatenate([p0a_ref[0], p0b_ref[0]], axis=0)
        b_s[...] = jnp.concatenate([p1a_ref[0], p1b_ref[0]], axis=0)
        c_s[...] = jnp.concatenate(
            [p2a_ref[0] * vals_ref[0], p2b_ref[0] * vals_ref[1]], axis=1)

    gs = vals_ref[0] + vals_ref[1]
    aw = a_s[...]
    bw = b_s[...]
    cw = c_s[...]
    pw = proj_ref[...]
    for j in range(H_BLK):
        xt = x_ref[:, j, :]
        h = jnp.dot(aw, xt, preferred_element_type=jnp.float32)
        gg = jnp.dot(bw, s_ref[:, j, :], preferred_element_type=jnp.float32)
        g = gg * jax.nn.sigmoid(gg)
        u = jnp.dot(cw, h * g, preferred_element_type=jnp.float32)
        out_ref[:, j, :] = jnp.dot(pw, u + gs * xt,
                                   preferred_element_type=jnp.float32)


@functools.partial(jax.jit, static_argnames=("interpret",))
def kernel(x, shared, mlp_w1, mlp_b1, mlp_w2, mlp_b2, gate_w, freq_gate_w,
           p0, p1, p2, proj_out_w, interpret=False):
    f32 = jnp.float32
    x3 = x.reshape(DIM, H, W)

    fe_part, pooled_part = pl.pallas_call(
        _reduce_body,
        grid=(DIM // C_BLK,),
        in_specs=[pl.BlockSpec((C_BLK, H, W), lambda i: (i, 0, 0))],
        out_specs=[pl.BlockSpec((C_BLK, W), lambda i: (i, 0)),
                   pl.BlockSpec((C_BLK, W), lambda i: (i, 0))],
        out_shape=[jax.ShapeDtypeStruct((DIM, W), f32),
                   jax.ShapeDtypeStruct((DIM, W), f32)],
        interpret=interpret,
    )(x3)

    idxv, valsv = pl.pallas_call(
        _router_body,
        out_shape=[jax.ShapeDtypeStruct((1, 128), jnp.int32),
                   jax.ShapeDtypeStruct((1, 128), f32)],
        interpret=interpret,
    )(fe_part, pooled_part, mlp_w1, mlp_b1.reshape(2 * DIM, 1),
      mlp_w2, mlp_b2.reshape(DIM, 1), gate_w, freq_gate_w)

    idx = idxv[0, :K]
    vals = valsv[0, :K]

    s3 = shared.reshape(DIM, H, W)

    grid_spec = pltpu.PrefetchScalarGridSpec(
        num_scalar_prefetch=2,
        grid=(H // H_BLK,),
        in_specs=[
            pl.BlockSpec((DIM, H_BLK, W), lambda p, i, v: (0, p, 0)),
            pl.BlockSpec((DIM, H_BLK, W), lambda p, i, v: (0, p, 0)),
            pl.BlockSpec((1, RANK, DIM), lambda p, i, v: (i[0], 0, 0)),
            pl.BlockSpec((1, RANK, DIM), lambda p, i, v: (i[1], 0, 0)),
            pl.BlockSpec((1, RANK, DIM), lambda p, i, v: (i[0], 0, 0)),
            pl.BlockSpec((1, RANK, DIM), lambda p, i, v: (i[1], 0, 0)),
            pl.BlockSpec((1, DIM, RANK), lambda p, i, v: (i[0], 0, 0)),
            pl.BlockSpec((1, DIM, RANK), lambda p, i, v: (i[1], 0, 0)),
            pl.BlockSpec((DIM, DIM), lambda p, i, v: (0, 0)),
        ],
        out_specs=pl.BlockSpec((DIM, H_BLK, W), lambda p, i, v: (0, p, 0)),
        scratch_shapes=[
            pltpu.VMEM((K * RANK, DIM), f32),
            pltpu.VMEM((K * RANK, DIM), f32),
            pltpu.VMEM((DIM, K * RANK), f32),
        ],
    )

    out3 = pl.pallas_call(
        _main_body,
        grid_spec=grid_spec,
        out_shape=jax.ShapeDtypeStruct((DIM, H, W), f32),
        interpret=interpret,
    )(idx, vals, x3, s3, p0, p0, p1, p1, p2, p2, proj_out_w)

    return out3.reshape(1, DIM, H, W)
